# baseline (speedup 1.0000x reference)
import math

import jax
import jax.numpy as jnp
from jax.experimental import pallas as pl

_BLOCK_ROWS = 1024


def _pe_body(out_ref):
    i = pl.program_id(0)
    rows = out_ref.shape[1]
    d_model = out_ref.shape[2]
    p = (i * rows + jax.lax.broadcasted_iota(jnp.int32, (1, rows, d_model), 1)).astype(jnp.float32)
    l = jax.lax.broadcasted_iota(jnp.int32, (1, rows, d_model), 2)
    parity = (l % 2).astype(jnp.float32)
    freq = jnp.exp((l - (l % 2)).astype(jnp.float32) * (-math.log(10000.0) / d_model))
    out_ref[...] = jnp.sin(p * freq + parity * (math.pi / 2.0))


def kernel(x, pe):
    seq_len = x.shape[1]
    d_model = pe.shape[2]
    grid = (seq_len // _BLOCK_ROWS,)
    out_shape = jax.ShapeDtypeStruct((1, seq_len, d_model), pe.dtype)
    return pl.pallas_call(
        _pe_body,
        grid=grid,
        out_specs=pl.BlockSpec((1, _BLOCK_ROWS, d_model), lambda i: (0, i, 0)),
        out_shape=out_shape,
    )()
